# baseline (speedup 1.0000x reference)
import functools

import jax
import jax.numpy as jnp
from jax.experimental import pallas as pl
from jax.experimental.pallas import tpu as pltpu

_BM = 200
_BS = 2000
_CSTRIDE = 9
_CN = 6


def _fused_kernel(x_ref, w1_ref, w2_ref, adj_ref, o_ref, s_ref, t_ref, c_ref,
                  *, nb, ns):
    i = pl.program_id(0)

    @pl.when(i < ns)
    def _s_phase():
        w = jnp.dot(w1_ref[...], w2_ref[...],
                    preferred_element_type=jnp.float32)
        s_ref[pl.ds(i * _BS, _BS), :] = jnp.dot(
            x_ref[...], w, preferred_element_type=jnp.float32)

    p1 = i - ns
    cacheable = (p1 % _CSTRIDE == 0) & (p1 < _CN * _CSTRIDE)

    @pl.when((i >= ns) & (i < ns + nb))
    def _pass1():
        t_ref[pl.ds(p1 * _BM, _BM), :] = jnp.dot(
            adj_ref[...], s_ref[...], preferred_element_type=jnp.float32)

    @pl.when((i >= ns) & (i < ns + nb) & cacheable)
    def _fill_cache():
        slot = p1 // _CSTRIDE
        c_ref[pl.ds(slot * _BM, _BM), :] = adj_ref[...].astype(jnp.bfloat16)

    p2 = i - (ns + nb)
    cached2 = (p2 % _CSTRIDE == 0) & (p2 < _CN * _CSTRIDE)

    @pl.when((i >= ns + nb) & jnp.logical_not(cached2))
    def _pass2_stream():
        o_ref[...] = jnp.dot(adj_ref[...], t_ref[...],
                             preferred_element_type=jnp.float32)

    @pl.when((i >= ns + nb) & cached2)
    def _pass2_cached():
        slot = p2 // _CSTRIDE
        blk = c_ref[pl.ds(slot * _BM, _BM), :]
        o_ref[...] = jnp.dot(blk, t_ref[...].astype(jnp.bfloat16),
                             preferred_element_type=jnp.float32)


def kernel(adj, x, W_enc, W_mean):
    n, d_in = x.shape
    d_emb = W_mean.shape[1]
    nb = n // _BM
    ns = n // _BS
    body = functools.partial(_fused_kernel, nb=nb, ns=ns)

    def adj_map(i):
        p1 = jnp.clip(i - ns, 0, nb - 1)
        p2 = i - (ns + nb)
        cached2 = (p2 % _CSTRIDE == 0) & (p2 < _CN * _CSTRIDE)
        p2 = jnp.where(cached2, p2 + 1, p2)
        return (jnp.where(i < ns + nb, p1, p2), 0)

    return pl.pallas_call(
        body,
        grid=(ns + 2 * nb,),
        in_specs=[
            pl.BlockSpec((_BS, d_in), lambda i: (jnp.minimum(i, ns - 1), 0)),
            pl.BlockSpec((d_in, d_emb), lambda i: (0, 0)),
            pl.BlockSpec((d_emb, d_emb), lambda i: (0, 0)),
            pl.BlockSpec((_BM, n), adj_map),
        ],
        out_specs=pl.BlockSpec(
            (_BM, d_emb), lambda i: (jnp.maximum(i - (ns + nb), 0), 0)),
        out_shape=jax.ShapeDtypeStruct((n, d_emb), jnp.float32),
        scratch_shapes=[
            pltpu.VMEM((n, d_emb), jnp.float32),
            pltpu.VMEM((n, d_emb), jnp.float32),
            pltpu.VMEM((_CN * _BM, n), jnp.bfloat16),
        ],
    )(x, W_enc, W_mean, adj)


# speedup vs baseline: 1.0093x; 1.0093x over previous
import functools

import jax
import jax.numpy as jnp
from jax.experimental import pallas as pl
from jax.experimental.pallas import tpu as pltpu

_BM = 200
_BS = 2000
_CN = 8


def _fused_kernel(x_ref, w1_ref, w2_ref, adj_ref, o_ref, oc_ref,
                  s_ref, t_ref, c_ref, *, nb, ns):
    i = pl.program_id(0)
    nu = nb - _CN

    @pl.when(i < ns)
    def _s_phase():
        w = jnp.dot(w1_ref[...], w2_ref[...],
                    preferred_element_type=jnp.float32)
        s = jnp.dot(x_ref[...], w, preferred_element_type=jnp.float32)
        s_ref[pl.ds(i * _BS, _BS), :] = s.astype(jnp.bfloat16)

    p1 = i - ns

    @pl.when((i >= ns) & (i < ns + nb))
    def _pass1():
        ab = adj_ref[...].astype(jnp.bfloat16)
        t = jnp.dot(ab, s_ref[...], preferred_element_type=jnp.float32)
        t_ref[pl.ds(p1 * _BM, _BM), :] = t.astype(jnp.bfloat16)

        @pl.when(p1 >= nu)
        def _fill_cache():
            c_ref[pl.ds((p1 - nu) * _BM, _BM), :] = ab

    j = i - (ns + nb)

    @pl.when(i >= ns + nb)
    def _pass2():
        ab = adj_ref[...].astype(jnp.bfloat16)
        o_ref[...] = jnp.dot(ab, t_ref[...],
                             preferred_element_type=jnp.float32)

    heavy = (j >= nu - 2 * _CN) & ((j - (nu - 2 * _CN)) % 2 == 0)

    @pl.when((i >= ns + nb) & heavy)
    def _pass2_cached():
        slot = (j - (nu - 2 * _CN)) // 2
        blk = c_ref[pl.ds(slot * _BM, _BM), :]
        oc_ref[...] = jnp.dot(blk, t_ref[...],
                              preferred_element_type=jnp.float32)


def kernel(adj, x, W_enc, W_mean):
    n, d_in = x.shape
    d_emb = W_mean.shape[1]
    nb = n // _BM
    ns = n // _BS
    nu = nb - _CN
    body = functools.partial(_fused_kernel, nb=nb, ns=ns)

    def adj_map(i):
        p1 = jnp.clip(i - ns, 0, nb - 1)
        p2 = jnp.clip(i - (ns + nb), 0, nu - 1)
        return (jnp.where(i < ns + nb, p1, p2), 0)

    def oc_map(i):
        j = i - (ns + nb)
        return (jnp.clip((j - (nu - 2 * _CN)) // 2, 0, _CN - 1), 0)

    out_main, out_cached = pl.pallas_call(
        body,
        grid=(ns + nb + nu,),
        in_specs=[
            pl.BlockSpec((_BS, d_in), lambda i: (jnp.minimum(i, ns - 1), 0)),
            pl.BlockSpec((d_in, d_emb), lambda i: (0, 0)),
            pl.BlockSpec((d_emb, d_emb), lambda i: (0, 0)),
            pl.BlockSpec((_BM, n), adj_map),
        ],
        out_specs=[
            pl.BlockSpec(
                (_BM, d_emb),
                lambda i: (jnp.clip(i - (ns + nb), 0, nu - 1), 0)),
            pl.BlockSpec((_BM, d_emb), oc_map),
        ],
        out_shape=[
            jax.ShapeDtypeStruct((nu * _BM, d_emb), jnp.float32),
            jax.ShapeDtypeStruct((_CN * _BM, d_emb), jnp.float32),
        ],
        scratch_shapes=[
            pltpu.VMEM((n, d_emb), jnp.bfloat16),
            pltpu.VMEM((n, d_emb), jnp.bfloat16),
            pltpu.VMEM((_CN * _BM, n), jnp.bfloat16),
        ],
        compiler_params=pltpu.CompilerParams(
            vmem_limit_bytes=64 * 1024 * 1024),
    )(x, W_enc, W_mean, adj)
    return jnp.concatenate([out_main, out_cached], axis=0)


# speedup vs baseline: 1.0258x; 1.0163x over previous
import functools

import jax
import jax.numpy as jnp
from jax.experimental import pallas as pl
from jax.experimental.pallas import tpu as pltpu

_BM = 200
_BS = 2000
_CN = 7
_HS = 6


def _fused_kernel(x_ref, w1_ref, w2_ref, adj_ref, o_ref, oc_ref,
                  s_ref, t_ref, t2_ref, c_ref, *, nb, ns):
    i = pl.program_id(0)
    nu = nb - _CN

    @pl.when(i < ns)
    def _s_phase():
        w = jnp.dot(w1_ref[...], w2_ref[...],
                    preferred_element_type=jnp.float32)
        s_ref[pl.ds(i * _BS, _BS), :] = jnp.dot(
            x_ref[...], w, preferred_element_type=jnp.float32)

    p1 = i - ns

    @pl.when((i >= ns) & (i < ns + nb))
    def _pass1():
        t = jnp.dot(adj_ref[...], s_ref[...],
                    preferred_element_type=jnp.float32)
        t_ref[pl.ds(p1 * _BM, _BM), :] = t
        t2_ref[pl.ds(p1 * _BM, _BM), :] = t.astype(jnp.bfloat16)

        @pl.when(p1 >= nu)
        def _fill_cache():
            c_ref[pl.ds((p1 - nu) * _BM, _BM), :] = (
                adj_ref[...].astype(jnp.bfloat16))

    j = i - (ns + nb)

    @pl.when(i >= ns + nb)
    def _pass2():
        o_ref[...] = jnp.dot(adj_ref[...], t_ref[...],
                             preferred_element_type=jnp.float32)

    heavy = (j % _HS == 0) & (j < _CN * _HS)

    @pl.when((i >= ns + nb) & heavy)
    def _pass2_cached():
        slot = j // _HS
        blk = c_ref[pl.ds(slot * _BM, _BM), :]
        oc_ref[...] = jnp.dot(blk, t2_ref[...],
                              preferred_element_type=jnp.float32)


def kernel(adj, x, W_enc, W_mean):
    n, d_in = x.shape
    d_emb = W_mean.shape[1]
    nb = n // _BM
    ns = n // _BS
    nu = nb - _CN
    body = functools.partial(_fused_kernel, nb=nb, ns=ns)

    def adj_map(i):
        p1 = jnp.clip(i - ns, 0, nb - 1)
        p2 = jnp.clip(i - (ns + nb), 0, nu - 1)
        return (jnp.where(i < ns + nb, p1, p2), 0)

    def oc_map(i):
        j = i - (ns + nb)
        return (jnp.clip(j // _HS, 0, _CN - 1), 0)

    out_main, out_cached = pl.pallas_call(
        body,
        grid=(ns + nb + nu,),
        in_specs=[
            pl.BlockSpec((_BS, d_in), lambda i: (jnp.minimum(i, ns - 1), 0)),
            pl.BlockSpec((d_in, d_emb), lambda i: (0, 0)),
            pl.BlockSpec((d_emb, d_emb), lambda i: (0, 0)),
            pl.BlockSpec((_BM, n), adj_map),
        ],
        out_specs=[
            pl.BlockSpec(
                (_BM, d_emb),
                lambda i: (jnp.clip(i - (ns + nb), 0, nu - 1), 0)),
            pl.BlockSpec((_BM, d_emb), oc_map),
        ],
        out_shape=[
            jax.ShapeDtypeStruct((nu * _BM, d_emb), jnp.float32),
            jax.ShapeDtypeStruct((_CN * _BM, d_emb), jnp.float32),
        ],
        scratch_shapes=[
            pltpu.VMEM((n, d_emb), jnp.float32),
            pltpu.VMEM((n, d_emb), jnp.float32),
            pltpu.VMEM((n, d_emb), jnp.bfloat16),
            pltpu.VMEM((_CN * _BM, n), jnp.bfloat16),
        ],
        compiler_params=pltpu.CompilerParams(
            vmem_limit_bytes=64 * 1024 * 1024),
    )(x, W_enc, W_mean, adj)
    return jnp.concatenate([out_main, out_cached], axis=0)


# speedup vs baseline: 1.0383x; 1.0122x over previous
import functools

import jax
import jax.numpy as jnp
from jax.experimental import pallas as pl
from jax.experimental.pallas import tpu as pltpu

_BM = 200
_BS = 2000
_CN = 7


def _fused_kernel(x_ref, w1_ref, w2_ref, adj_ref, o_ref,
                  s_ref, t_ref, t2_ref, c_ref, *, nb, ns):
    i = pl.program_id(0)
    nu = nb - _CN

    @pl.when(i < ns)
    def _s_phase():
        w = jnp.dot(w1_ref[...], w2_ref[...],
                    preferred_element_type=jnp.float32)
        s_ref[pl.ds(i * _BS, _BS), :] = jnp.dot(
            x_ref[...], w, preferred_element_type=jnp.float32)

    p1 = i - ns

    @pl.when((i >= ns) & (i < ns + nb))
    def _pass1():
        t = jnp.dot(adj_ref[...], s_ref[...],
                    preferred_element_type=jnp.float32)
        t_ref[pl.ds(p1 * _BM, _BM), :] = t
        t2_ref[pl.ds(p1 * _BM, _BM), :] = t.astype(jnp.bfloat16)

        @pl.when(p1 >= nu)
        def _fill_cache():
            c_ref[pl.ds((p1 - nu) * _BM, _BM), :] = (
                adj_ref[...].astype(jnp.bfloat16))

    j = i - (ns + nb)

    @pl.when((i >= ns + nb) & (j < nu))
    def _pass2_stream():
        o_ref[...] = jnp.dot(adj_ref[...], t_ref[...],
                             preferred_element_type=jnp.float32)

    @pl.when(j >= nu)
    def _pass2_cached():
        slot = j - nu
        blk = c_ref[pl.ds(slot * _BM, _BM), :]
        o_ref[...] = jnp.dot(blk, t2_ref[...],
                             preferred_element_type=jnp.float32)


def kernel(adj, x, W_enc, W_mean):
    n, d_in = x.shape
    d_emb = W_mean.shape[1]
    nb = n // _BM
    ns = n // _BS
    nu = nb - _CN
    body = functools.partial(_fused_kernel, nb=nb, ns=ns)

    def adj_map(i):
        p1 = jnp.clip(i - ns, 0, nb - 1)
        p2 = jnp.clip(i - (ns + nb), 0, nu - 1)
        return (jnp.where(i < ns + nb, p1, p2), 0)

    return pl.pallas_call(
        body,
        grid=(ns + 2 * nb,),
        in_specs=[
            pl.BlockSpec((_BS, d_in), lambda i: (jnp.minimum(i, ns - 1), 0)),
            pl.BlockSpec((d_in, d_emb), lambda i: (0, 0)),
            pl.BlockSpec((d_emb, d_emb), lambda i: (0, 0)),
            pl.BlockSpec((_BM, n), adj_map),
        ],
        out_specs=pl.BlockSpec(
            (_BM, d_emb), lambda i: (jnp.maximum(i - (ns + nb), 0), 0)),
        out_shape=jax.ShapeDtypeStruct((n, d_emb), jnp.float32),
        scratch_shapes=[
            pltpu.VMEM((n, d_emb), jnp.float32),
            pltpu.VMEM((n, d_emb), jnp.float32),
            pltpu.VMEM((n, d_emb), jnp.bfloat16),
            pltpu.VMEM((_CN * _BM, n), jnp.bfloat16),
        ],
        compiler_params=pltpu.CompilerParams(
            vmem_limit_bytes=64 * 1024 * 1024),
    )(x, W_enc, W_mean, adj)


# speedup vs baseline: 1.0471x; 1.0086x over previous
import functools

import jax
import jax.numpy as jnp
from jax.experimental import pallas as pl
from jax.experimental.pallas import tpu as pltpu

_BM = 200
_BS = 2000
_CN = 7
_KC = 2560
_NC = 4
_NP = 10240


def _fused_kernel(x_ref, w1_ref, w2_ref, adj_ref, o_ref,
                  s_ref, t_ref, t2_ref, c_ref, acc_ref, *, nb, ns):
    i = pl.program_id(0)
    nu = nb - _CN

    @pl.when(i < ns)
    def _s_phase():
        w = jnp.dot(w1_ref[...], w2_ref[...],
                    preferred_element_type=jnp.float32)
        s_ref[pl.ds(i * _BS, _BS), :] = jnp.dot(
            x_ref[...], w, preferred_element_type=jnp.float32)
        rows = (_CN * _BM) // ns
        c_ref[pl.ds(i * rows, rows), :] = jnp.zeros(
            (rows, _NP), jnp.bfloat16)

    @pl.when(i == 0)
    def _t2_pad():
        pad = _NP - _BM * nb
        t2_ref[pl.ds(_BM * nb, pad), :] = jnp.zeros(
            (pad, t2_ref.shape[1]), jnp.bfloat16)

    p1 = i - ns

    @pl.when((i >= ns) & (i < ns + nb))
    def _pass1():
        t = jnp.dot(adj_ref[...], s_ref[...],
                    preferred_element_type=jnp.float32)
        t_ref[pl.ds(p1 * _BM, _BM), :] = t
        t2_ref[pl.ds(p1 * _BM, _BM), :] = t.astype(jnp.bfloat16)

        @pl.when(p1 >= nu)
        def _fill_cache():
            c_ref[pl.ds((p1 - nu) * _BM, _BM), :adj_ref.shape[1]] = (
                adj_ref[...].astype(jnp.bfloat16))

    j = i - (ns + nb)

    @pl.when((i >= ns + nb) & (j < nu))
    def _pass2_stream():
        o_ref[...] = jnp.dot(adj_ref[...], t_ref[...],
                             preferred_element_type=jnp.float32)

    slot = j // _NC
    for k in range(_NC):
        @pl.when((i >= ns + nb) & (j < _CN * _NC) & (j % _NC == k))
        def _piggyback(k=k):
            blk = c_ref[pl.ds(slot * _BM, _BM), k * _KC:(k + 1) * _KC]
            part = jnp.dot(blk, t2_ref[pl.ds(k * _KC, _KC), :],
                           preferred_element_type=jnp.float32)
            if k == 0:
                acc_ref[pl.ds(slot * _BM, _BM), :] = part
            else:
                acc_ref[pl.ds(slot * _BM, _BM), :] = (
                    acc_ref[pl.ds(slot * _BM, _BM), :] + part)

    @pl.when(j >= nu)
    def _pass2_cached():
        o_ref[...] = acc_ref[pl.ds((j - nu) * _BM, _BM), :]


def kernel(adj, x, W_enc, W_mean):
    n, d_in = x.shape
    d_emb = W_mean.shape[1]
    nb = n // _BM
    ns = n // _BS
    nu = nb - _CN
    body = functools.partial(_fused_kernel, nb=nb, ns=ns)

    def adj_map(i):
        p1 = jnp.clip(i - ns, 0, nb - 1)
        p2 = jnp.clip(i - (ns + nb), 0, nu - 1)
        return (jnp.where(i < ns + nb, p1, p2), 0)

    return pl.pallas_call(
        body,
        grid=(ns + 2 * nb,),
        in_specs=[
            pl.BlockSpec((_BS, d_in), lambda i: (jnp.minimum(i, ns - 1), 0)),
            pl.BlockSpec((d_in, d_emb), lambda i: (0, 0)),
            pl.BlockSpec((d_emb, d_emb), lambda i: (0, 0)),
            pl.BlockSpec((_BM, n), adj_map),
        ],
        out_specs=pl.BlockSpec(
            (_BM, d_emb), lambda i: (jnp.maximum(i - (ns + nb), 0), 0)),
        out_shape=jax.ShapeDtypeStruct((n, d_emb), jnp.float32),
        scratch_shapes=[
            pltpu.VMEM((n, d_emb), jnp.float32),
            pltpu.VMEM((n, d_emb), jnp.float32),
            pltpu.VMEM((_NP, d_emb), jnp.bfloat16),
            pltpu.VMEM((_CN * _BM, _NP), jnp.bfloat16),
            pltpu.VMEM((_CN * _BM, d_emb), jnp.float32),
        ],
        compiler_params=pltpu.CompilerParams(
            vmem_limit_bytes=64 * 1024 * 1024),
    )(x, W_enc, W_mean, adj)
